# baseline (speedup 1.0000x reference)
import jax
import jax.numpy as jnp
from jax.experimental import pallas as pl
from jax.experimental.pallas import tpu as pltpu
from jax.sharding import Mesh, PartitionSpec as P
from jax.experimental.shard_map import shard_map

_D_MODEL = 1024
_D_FFN = 4096
_TOP_K = 256
_BLK = 256


def _ffn_kernel(x_ref, wg_ref, wu_ref, wd_ref, o_ref):
    x = x_ref[...]
    g = jnp.dot(x, wg_ref[...], preferred_element_type=jnp.float32)

    bits = jax.lax.bitcast_convert_type(g, jnp.int32)
    key = bits ^ ((bits >> 31) & jnp.int32(0x7FFFFFFF))

    cnt_pos = jnp.sum((key >= 0).astype(jnp.int32), axis=1, keepdims=True)
    t = jnp.where(cnt_pos >= _TOP_K, jnp.int32(0), jnp.int32(-(2**31)))
    for b in range(30, -1, -1):
        cand = t | jnp.int32(1 << b)
        cnt = jnp.sum((key >= cand).astype(jnp.int32), axis=1, keepdims=True)
        t = jnp.where(cnt >= _TOP_K, cand, t)
    mask = key >= t

    u = jnp.dot(x.astype(jnp.bfloat16), wu_ref[...],
                preferred_element_type=jnp.float32)
    z = jnp.where(mask, g * jax.nn.sigmoid(g) * u, 0.0)
    o_ref[...] = jnp.dot(z.astype(jnp.bfloat16), wd_ref[...],
                         preferred_element_type=jnp.float32)


def _run(x2, w_gate, wu, wd):
    n = x2.shape[0]
    return pl.pallas_call(
        _ffn_kernel,
        grid=(n // _BLK,),
        in_specs=[
            pl.BlockSpec((_BLK, _D_MODEL), lambda i: (i, 0)),
            pl.BlockSpec((_D_MODEL, _D_FFN), lambda i: (0, 0)),
            pl.BlockSpec((_D_MODEL, _D_FFN), lambda i: (0, 0)),
            pl.BlockSpec((_D_FFN, _D_MODEL), lambda i: (0, 0)),
        ],
        out_specs=pl.BlockSpec((_BLK, _D_MODEL), lambda i: (i, 0)),
        out_shape=jax.ShapeDtypeStruct((n, _D_MODEL), jnp.float32),
        compiler_params=pltpu.CompilerParams(
            dimension_semantics=("arbitrary",),
        ),
    )(x2, w_gate, wu, wd)


def kernel(x, w_gate, w_up, w_down):
    orig_shape = x.shape
    x2 = x.reshape(-1, _D_MODEL)
    n = x2.shape[0]
    wu = w_up.astype(jnp.bfloat16)
    wd = w_down.astype(jnp.bfloat16)
    devs = jax.devices()
    nd = 1
    for cand in (4, 2):
        if len(devs) >= cand and n % (cand * _BLK) == 0:
            nd = cand
            break
    if nd > 1:
        mesh = Mesh(devs[:nd], ("d",))
        fn = shard_map(
            _run, mesh=mesh,
            in_specs=(P("d", None), P(None, None), P(None, None), P(None, None)),
            out_specs=P("d", None), check_rep=False,
        )
        out = fn(x2, w_gate, wu, wd)
    else:
        out = _run(x2, w_gate, wu, wd)
    return out.reshape(orig_shape)


# speedup vs baseline: 2.2565x; 2.2565x over previous
import jax
import jax.numpy as jnp
from jax.experimental import pallas as pl
from jax.experimental.pallas import tpu as pltpu

_D_MODEL = 1024
_D_FFN = 4096
_TOP_K = 256
_B = 128
_NBLK = 2048 // _B


def _threshold_mask(g):
    bits = jax.lax.bitcast_convert_type(g, jnp.int32)
    key = bits ^ ((bits >> 31) & jnp.int32(0x7FFFFFFF))
    cnt_pos = jnp.sum((key >= 0).astype(jnp.int32), axis=1, keepdims=True)
    t = jnp.where(cnt_pos >= _TOP_K, jnp.int32(0), jnp.int32(-(2**31)))
    for b in range(30, -1, -1):
        cand = t | jnp.int32(1 << b)
        cnt = jnp.sum((key >= cand).astype(jnp.int32), axis=1, keepdims=True)
        t = jnp.where(cnt >= _TOP_K, cand, t)
    return key >= t


def _ffn_kernel(xa_ref, xb_ref, xc_ref, wg_ref, wu_ref, wd_ref,
                oe_ref, oo_ref, ga_ref, gb_ref):
    wg = wg_ref[...]
    ga_ref[...] = jnp.dot(xa_ref[...], wg, preferred_element_type=jnp.float32)

    g_b = gb_ref[...]
    mask_b = _threshold_mask(g_b)
    u_b = jnp.dot(xc_ref[...].astype(jnp.bfloat16), wu_ref[...],
                  preferred_element_type=jnp.float32)
    z_b = jnp.where(mask_b, g_b * jax.nn.sigmoid(g_b) * u_b, 0.0)
    oo_ref[...] = jnp.dot(z_b.astype(jnp.bfloat16), wd_ref[...],
                          preferred_element_type=jnp.float32)

    gb_ref[...] = jnp.dot(xb_ref[...], wg, preferred_element_type=jnp.float32)

    g_a = ga_ref[...]
    mask_a = _threshold_mask(g_a)
    u_a = jnp.dot(xa_ref[...].astype(jnp.bfloat16), wu_ref[...],
                  preferred_element_type=jnp.float32)
    z_a = jnp.where(mask_a, g_a * jax.nn.sigmoid(g_a) * u_a, 0.0)
    oe_ref[...] = jnp.dot(z_a.astype(jnp.bfloat16), wd_ref[...],
                          preferred_element_type=jnp.float32)


def kernel(x, w_gate, w_up, w_down):
    orig_shape = x.shape
    x2 = x.reshape(-1, _D_MODEL)
    wu = w_up.astype(jnp.bfloat16)
    wd = w_down.astype(jnp.bfloat16)
    half = _NBLK // 2
    nsteps = half + 1
    out_e, out_o = pl.pallas_call(
        _ffn_kernel,
        grid=(nsteps,),
        in_specs=[
            pl.BlockSpec((_B, _D_MODEL),
                         lambda i: (jnp.minimum(2 * i, _NBLK - 2), 0)),
            pl.BlockSpec((_B, _D_MODEL),
                         lambda i: (jnp.minimum(2 * i + 1, _NBLK - 1), 0)),
            pl.BlockSpec((_B, _D_MODEL),
                         lambda i: (jnp.clip(2 * i - 1, 0, _NBLK - 1), 0)),
            pl.BlockSpec((_D_MODEL, _D_FFN), lambda i: (0, 0)),
            pl.BlockSpec((_D_MODEL, _D_FFN), lambda i: (0, 0)),
            pl.BlockSpec((_D_FFN, _D_MODEL), lambda i: (0, 0)),
        ],
        out_specs=[
            pl.BlockSpec((_B, _D_MODEL),
                         lambda i: (jnp.minimum(i, _NBLK // 2 - 1), 0)),
            pl.BlockSpec((_B, _D_MODEL),
                         lambda i: (jnp.maximum(i - 1, 0), 0)),
        ],
        out_shape=[
            jax.ShapeDtypeStruct((half * _B, _D_MODEL), jnp.float32),
            jax.ShapeDtypeStruct((half * _B, _D_MODEL), jnp.float32),
        ],
        scratch_shapes=[
            pltpu.VMEM((_B, _D_FFN), jnp.float32),
            pltpu.VMEM((_B, _D_FFN), jnp.float32),
        ],
        compiler_params=pltpu.CompilerParams(
            dimension_semantics=("arbitrary",),
        ),
    )(x2, x2, x2, w_gate, wu, wd)
    out = jnp.stack(
        [out_e.reshape(half, _B, _D_MODEL), out_o.reshape(half, _B, _D_MODEL)],
        axis=1,
    ).reshape(-1, _D_MODEL)
    return out.reshape(orig_shape)


# speedup vs baseline: 2.8391x; 1.2582x over previous
import jax
import jax.numpy as jnp
from jax.experimental import pallas as pl
from jax.experimental.pallas import tpu as pltpu

_D_MODEL = 1024
_D_FFN = 4096
_TOP_K = 256
_BLK = 256


def _ffn_kernel(x_ref, wg_ref, wu_ref, wd_ref, o_ref):
    x = x_ref[...]
    g = jnp.dot(x, wg_ref[...], preferred_element_type=jnp.float32)
    u = jnp.dot(x.astype(jnp.bfloat16), wu_ref[...],
                preferred_element_type=jnp.float32)

    bits = jax.lax.bitcast_convert_type(g, jnp.int32)
    key = bits ^ ((bits >> 31) & jnp.int32(0x7FFFFFFF))

    cnt_pos = jnp.sum((key >= 0).astype(jnp.int32), axis=1, keepdims=True)
    t = jnp.where(cnt_pos >= _TOP_K, jnp.int32(0), jnp.int32(-(2**31)))
    for b in range(30, -1, -1):
        cand = t | jnp.int32(1 << b)
        cnt = jnp.sum((key >= cand).astype(jnp.int32), axis=1, keepdims=True)
        t = jnp.where(cnt >= _TOP_K, cand, t)
    mask = key >= t

    z = jnp.where(mask, g * jax.nn.sigmoid(g) * u, 0.0)
    o_ref[...] = jnp.dot(z.astype(jnp.bfloat16), wd_ref[...],
                         preferred_element_type=jnp.float32)


def kernel(x, w_gate, w_up, w_down):
    orig_shape = x.shape
    x2 = x.reshape(-1, _D_MODEL)
    n = x2.shape[0]
    wu = w_up.astype(jnp.bfloat16)
    wd = w_down.astype(jnp.bfloat16)
    out = pl.pallas_call(
        _ffn_kernel,
        grid=(n // _BLK,),
        in_specs=[
            pl.BlockSpec((_BLK, _D_MODEL), lambda i: (i, 0)),
            pl.BlockSpec((_D_MODEL, _D_FFN), lambda i: (0, 0)),
            pl.BlockSpec((_D_MODEL, _D_FFN), lambda i: (0, 0)),
            pl.BlockSpec((_D_FFN, _D_MODEL), lambda i: (0, 0)),
        ],
        out_specs=pl.BlockSpec((_BLK, _D_MODEL), lambda i: (i, 0)),
        out_shape=jax.ShapeDtypeStruct((n, _D_MODEL), jnp.float32),
        compiler_params=pltpu.CompilerParams(
            dimension_semantics=("arbitrary",),
        ),
    )(x2, w_gate, wu, wd)
    return out.reshape(orig_shape)
